# baseline (speedup 1.0000x reference)
import functools

import jax
import jax.numpy as jnp
from jax import lax
from jax.experimental import pallas as pl
from jax.experimental.pallas import tpu as pltpu
from jax.experimental.pallas import tpu_sc as plsc


@functools.lru_cache(maxsize=None)
def _build_gather(n_tok: int, n_batch: int, dim: int):
    info = plsc.get_sparse_core_info()
    nlanes = info.num_lanes
    nw = info.num_cores * info.num_subcores
    assert n_batch % nw == 0
    chunk = n_batch // nw
    assert chunk % nlanes == 0

    mesh = plsc.VectorSubcoreMesh(core_axis_name="c", subcore_axis_name="s")

    @functools.partial(
        pl.kernel,
        mesh=mesh,
        out_type=jax.ShapeDtypeStruct((n_tok, dim, n_batch), jnp.float32),
        scratch_types=[
            pltpu.VMEM((n_tok, chunk), jnp.int32),
            pltpu.VMEM((2, chunk, dim), jnp.float32),
            pltpu.VMEM((2, dim, chunk), jnp.float32),
            pltpu.SemaphoreType.DMA((2,)),
            pltpu.SemaphoreType.DMA((2,)),
        ],
        compiler_params=pltpu.CompilerParams(
            use_tc_tiling_on_sc=False, needs_layout_passes=False
        ),
    )
    def gather_kernel(idx_hbm, table_hbm, out_hbm, idx_v, rows_v, tbuf_v, gsem, ssem):
        wid = lax.axis_index("s") * info.num_cores + lax.axis_index("c")
        b0 = pl.multiple_of(wid * chunk, 128)

        pltpu.sync_copy(idx_hbm.at[:, pl.ds(b0, chunk)], idx_v)

        def fire_gather(t, buf):
            pltpu.async_copy(
                table_hbm.at[idx_v.at[t]], rows_v.at[buf], gsem.at[buf]
            )

        def wait_gather(buf):
            pltpu.make_async_copy(
                table_hbm.at[pl.ds(0, chunk)], rows_v.at[buf], gsem.at[buf]
            ).wait()

        def transpose(buf):
            lanes = lax.iota(jnp.int32, nlanes)

            @plsc.parallel_loop(0, chunk // nlanes, unroll=8)
            def _(i):
                row_idx = lanes + i * nlanes
                for d in range(dim):
                    col_idx = jnp.full((nlanes,), d, jnp.int32)
                    vals = plsc.load_gather(rows_v.at[buf], [row_idx, col_idx])
                    tbuf_v[buf, d, pl.ds(i * nlanes, nlanes)] = vals

        def fire_store(t, buf):
            pltpu.async_copy(
                tbuf_v.at[buf], out_hbm.at[t, :, pl.ds(b0, chunk)], ssem.at[buf]
            )

        def wait_store(buf):
            pltpu.make_async_copy(
                tbuf_v.at[buf], out_hbm.at[0, :, pl.ds(0, chunk)], ssem.at[buf]
            ).wait()

        assert n_tok >= 4 and n_tok % 2 == 0

        for b in range(2):
            fire_gather(b, b)
        for b in range(2):
            wait_gather(b)
            transpose(b)
            fire_store(b, b)
            fire_gather(b + 2, b)

        @pl.loop(0, (n_tok - 4) // 2)
        def _(i):
            for b in range(2):
                t = 2 + 2 * i + b
                wait_gather(b)
                wait_store(b)
                transpose(b)
                fire_store(t, b)
                fire_gather(t + 2, b)

        for b in range(2):
            wait_gather(b)
            wait_store(b)
            transpose(b)
            fire_store(n_tok - 2 + b, b)
        for b in range(2):
            wait_store(b)

    return gather_kernel


def kernel(indices, table):
    b, t = indices.shape
    dim = table.shape[1]
    idx_t = jnp.swapaxes(indices, 0, 1).astype(jnp.int32)
    out = _build_gather(t, b, dim)(idx_t, table)
    return jnp.transpose(out, (2, 0, 1))


# speedup vs baseline: 1.0124x; 1.0124x over previous
import functools

import jax
import jax.numpy as jnp
from jax import lax
from jax.experimental import pallas as pl
from jax.experimental.pallas import tpu as pltpu
from jax.experimental.pallas import tpu_sc as plsc


@functools.lru_cache(maxsize=None)
def _build_gather(n_tok: int, n_batch: int, dim: int):
    info = plsc.get_sparse_core_info()
    nlanes = info.num_lanes
    nw = info.num_cores * info.num_subcores
    assert n_batch % nw == 0
    chunk = n_batch // nw
    assert chunk % nlanes == 0

    mesh = plsc.VectorSubcoreMesh(core_axis_name="c", subcore_axis_name="s")

    @functools.partial(
        pl.kernel,
        mesh=mesh,
        out_type=jax.ShapeDtypeStruct((n_tok, dim, n_batch), jnp.float32),
        scratch_types=[
            pltpu.VMEM((n_tok, chunk), jnp.int32),
            pltpu.VMEM((2, chunk, dim), jnp.float32),
            pltpu.VMEM((2, dim, chunk), jnp.float32),
            pltpu.SemaphoreType.DMA((2,)),
            pltpu.SemaphoreType.DMA((2,)),
        ],
        compiler_params=pltpu.CompilerParams(
            use_tc_tiling_on_sc=False, needs_layout_passes=False
        ),
    )
    def gather_kernel(idx_hbm, table_hbm, out_hbm, idx_v, rows_v, tbuf_v, gsem, ssem):
        wid = lax.axis_index("s") * info.num_cores + lax.axis_index("c")
        b0 = pl.multiple_of(wid * chunk, 128)

        pltpu.sync_copy(idx_hbm.at[:, pl.ds(b0, chunk)], idx_v)

        def fire_gather(t, buf):
            pltpu.async_copy(
                table_hbm.at[idx_v.at[t]], rows_v.at[buf], gsem.at[buf]
            )

        def wait_gather(buf):
            pltpu.make_async_copy(
                table_hbm.at[pl.ds(0, chunk)], rows_v.at[buf], gsem.at[buf]
            ).wait()

        def transpose(buf):
            lanes = lax.iota(jnp.int32, nlanes)

            @plsc.parallel_loop(0, chunk // nlanes, unroll=4)
            def _(i):
                row_idx = lanes + i * nlanes
                for d in range(dim):
                    col_idx = jnp.full((nlanes,), d, jnp.int32)
                    vals = plsc.load_gather(rows_v.at[buf], [row_idx, col_idx])
                    tbuf_v[buf, d, pl.ds(i * nlanes, nlanes)] = vals

        def fire_store(t, buf):
            pltpu.async_copy(
                tbuf_v.at[buf], out_hbm.at[t, :, pl.ds(b0, chunk)], ssem.at[buf]
            )

        def wait_store(buf):
            pltpu.make_async_copy(
                tbuf_v.at[buf], out_hbm.at[0, :, pl.ds(0, chunk)], ssem.at[buf]
            ).wait()

        assert n_tok >= 4 and n_tok % 2 == 0

        for b in range(2):
            fire_gather(b, b)
        for b in range(2):
            wait_gather(b)
            transpose(b)
            fire_store(b, b)
            fire_gather(b + 2, b)

        @pl.loop(0, (n_tok - 4) // 2)
        def _(i):
            for b in range(2):
                t = 2 + 2 * i + b
                wait_gather(b)
                wait_store(b)
                transpose(b)
                fire_store(t, b)
                fire_gather(t + 2, b)

        for b in range(2):
            wait_gather(b)
            wait_store(b)
            transpose(b)
            fire_store(n_tok - 2 + b, b)
        for b in range(2):
            wait_store(b)

    return gather_kernel


def kernel(indices, table):
    b, t = indices.shape
    dim = table.shape[1]
    idx_t = jnp.swapaxes(indices, 0, 1).astype(jnp.int32)
    out = _build_gather(t, b, dim)(idx_t, table)
    return jnp.transpose(out, (2, 0, 1))
